# baseline (speedup 1.0000x reference)
import functools

import jax
import jax.numpy as jnp
from jax import lax
from jax.experimental import pallas as pl
from jax.experimental.pallas import tpu as pltpu
from jax.experimental.pallas import tpu_sc as plsc

NC = 2
NS = 16
NW = NC * NS
CHUNK = 256


def _gather_sc(table_pad, idx_flat):
    n = idx_flat.shape[0]
    mesh = plsc.VectorSubcoreMesh(core_axis_name="c", subcore_axis_name="s")
    idx2 = idx_flat.reshape(1, n)

    @functools.partial(
        pl.kernel,
        mesh=mesh,
        out_type=jax.ShapeDtypeStruct((n, 128), jnp.float32),
    )
    def k(table_hbm, idx_hbm, out_hbm):
        def body(i_vmem, o_vmem):
            pltpu.sync_copy(table_hbm.at[i_vmem.at[0]], o_vmem)

        pltpu.emit_pipeline(
            body,
            grid=(n // CHUNK,),
            in_specs=[pl.BlockSpec((1, CHUNK), index_map=lambda i: (0, i))],
            out_specs=[pl.BlockSpec((CHUNK, 128), index_map=lambda i: (i, 0))],
            core_axis_name=("c", "s"),
            dimension_semantics=(pltpu.PARALLEL,),
        )(idx_hbm, out_hbm)

    return k(table_pad, idx2)


def _transpose_tc(emb, e):
    b, l, ep = emb.shape
    bb = 32

    def body(x_ref, o_ref):
        for i in range(bb):
            o_ref[i] = x_ref[i, :, 0:e].T

    return pl.pallas_call(
        body,
        grid=(b // bb,),
        in_specs=[pl.BlockSpec((bb, l, ep), lambda i: (i, 0, 0))],
        out_specs=pl.BlockSpec((bb, e, l), lambda i: (i, 0, 0)),
        out_shape=jax.ShapeDtypeStruct((b, e, l), jnp.float32),
    )(emb)


NCHUNKS = 4


def kernel(x, table):
    b, l = x.shape
    e = table.shape[1]
    idx = x.reshape(-1).astype(jnp.int32)
    table_pad = jnp.pad(table, ((0, 0), (0, 128 - e)))
    bc = b // NCHUNKS
    outs = []
    for c in range(NCHUNKS):
        emb = _gather_sc(table_pad, idx[c * bc * l:(c + 1) * bc * l])
        outs.append(_transpose_tc(emb.reshape(bc, l, 128), e))
    return jnp.concatenate(outs, axis=0)


# speedup vs baseline: 1.2052x; 1.2052x over previous
import functools

import jax
import jax.numpy as jnp
from jax import lax
from jax.experimental import pallas as pl
from jax.experimental.pallas import tpu as pltpu
from jax.experimental.pallas import tpu_sc as plsc

NC = 2
NS = 16
NW = NC * NS
CHUNK = 256


def _gather_sc(table_pad, idx_flat):
    n = idx_flat.shape[0]
    mesh = plsc.VectorSubcoreMesh(core_axis_name="c", subcore_axis_name="s")
    idx2 = idx_flat.reshape(1, n)

    @functools.partial(
        pl.kernel,
        mesh=mesh,
        out_type=jax.ShapeDtypeStruct((n, 128), jnp.float32),
    )
    def k(table_hbm, idx_hbm, out_hbm):
        def body(i_vmem, o_vmem):
            pltpu.sync_copy(table_hbm.at[i_vmem.at[0]], o_vmem)

        pltpu.emit_pipeline(
            body,
            grid=(n // CHUNK,),
            in_specs=[pl.BlockSpec((1, CHUNK), index_map=lambda i: (0, i))],
            out_specs=[pl.BlockSpec((CHUNK, 128), index_map=lambda i: (i, 0))],
            core_axis_name=("c", "s"),
            dimension_semantics=(pltpu.PARALLEL,),
        )(idx_hbm, out_hbm)

    return k(table_pad, idx2)


def _transpose_tc(emb, e):
    b, l, ep = emb.shape
    bb = 32

    def body(x_ref, o_ref):
        for i in range(bb):
            o_ref[i] = x_ref[i, :, 0:e].T

    return pl.pallas_call(
        body,
        grid=(b // bb,),
        in_specs=[pl.BlockSpec((bb, l, ep), lambda i: (i, 0, 0))],
        out_specs=pl.BlockSpec((bb, e, l), lambda i: (i, 0, 0)),
        out_shape=jax.ShapeDtypeStruct((b, e, l), jnp.float32),
    )(emb)


def kernel(x, table):
    b, l = x.shape
    e = table.shape[1]
    idx = x.reshape(-1).astype(jnp.int32)
    table_pad = jnp.pad(table, ((0, 0), (0, 128 - e)))
    emb = _gather_sc(table_pad, idx)
    return _transpose_tc(emb.reshape(b, l, 128), e)


# speedup vs baseline: 1.7853x; 1.4814x over previous
import functools

import jax
import jax.numpy as jnp
from jax import lax
from jax.experimental import pallas as pl
from jax.experimental.pallas import tpu as pltpu
from jax.experimental.pallas import tpu_sc as plsc

NC = 2
NS = 16
NW = NC * NS
CHUNK = 256


def _gather_sc(table_pad, idx_flat):
    n = idx_flat.shape[0]
    mesh = plsc.VectorSubcoreMesh(core_axis_name="c", subcore_axis_name="s")
    idx2 = idx_flat.reshape(1, n)

    @functools.partial(
        pl.kernel,
        mesh=mesh,
        out_type=jax.ShapeDtypeStruct((n, 128), jnp.float32),
    )
    def k(table_hbm, idx_hbm, out_hbm):
        def body(i_vmem, o_vmem):
            pltpu.sync_copy(table_hbm.at[i_vmem.at[0]], o_vmem)

        pltpu.emit_pipeline(
            body,
            grid=(n // CHUNK,),
            in_specs=[pl.BlockSpec((1, CHUNK), index_map=lambda i: (0, i))],
            out_specs=[pl.BlockSpec((CHUNK, 128), index_map=lambda i: (i, 0))],
            core_axis_name=("c", "s"),
            dimension_semantics=(pltpu.PARALLEL,),
        )(idx_hbm, out_hbm)

    return k(table_pad, idx2)


def _transpose_tc(emb, e):
    l, b, ep = emb.shape
    bb = 2048
    lb = 8

    def body(x_ref, o_ref):
        o_ref[...] = jnp.transpose(x_ref[:, :, 0:e], (2, 0, 1))

    return pl.pallas_call(
        body,
        grid=(b // bb, l // lb),
        in_specs=[pl.BlockSpec((lb, bb, ep), lambda i, j: (j, i, 0))],
        out_specs=pl.BlockSpec((e, lb, bb), lambda i, j: (0, j, i)),
        out_shape=jax.ShapeDtypeStruct((e, l, b), jnp.float32),
    )(emb)


def kernel(x, table):
    b, l = x.shape
    e = table.shape[1]
    idx = x.T.reshape(-1).astype(jnp.int32)
    table_pad = jnp.pad(table, ((0, 0), (0, 128 - e)))
    emb = _gather_sc(table_pad, idx)
    z = _transpose_tc(emb.reshape(l, b, 128), e)
    return jnp.transpose(z, (2, 0, 1))


# speedup vs baseline: 1.7869x; 1.0009x over previous
import functools

import jax
import jax.numpy as jnp
from jax import lax
from jax.experimental import pallas as pl
from jax.experimental.pallas import tpu as pltpu
from jax.experimental.pallas import tpu_sc as plsc

NC = 2
NS = 16
NW = NC * NS
CHUNK = 256


def _gather_sc(table_pad, idx_flat):
    n = idx_flat.shape[0]
    mesh = plsc.VectorSubcoreMesh(core_axis_name="c", subcore_axis_name="s")
    idx2 = idx_flat.reshape(1, n)

    @functools.partial(
        pl.kernel,
        mesh=mesh,
        out_type=jax.ShapeDtypeStruct((n, 128), jnp.float32),
    )
    def k(table_hbm, idx_hbm, out_hbm):
        def body(i_vmem, o_vmem):
            pltpu.sync_copy(table_hbm.at[i_vmem.at[0]], o_vmem)

        pltpu.emit_pipeline(
            body,
            grid=(n // CHUNK,),
            in_specs=[pl.BlockSpec((1, CHUNK), index_map=lambda i: (0, i))],
            out_specs=[pl.BlockSpec((CHUNK, 128), index_map=lambda i: (i, 0))],
            core_axis_name=("c", "s"),
            dimension_semantics=(pltpu.PARALLEL,),
        )(idx_hbm, out_hbm)

    return k(table_pad, idx2)


def _transpose_tc(emb, e, l_total, lg0, zprev=None):
    l, b, ep = emb.shape
    bb = 2048
    lb = 8
    out_spec = pl.BlockSpec((e, lb, bb), lambda i, j: (0, j + lg0, i))
    out_shape = jax.ShapeDtypeStruct((e, l_total, b), jnp.float32)
    in_spec = pl.BlockSpec((lb, bb, ep), lambda i, j: (j, i, 0))
    grid = (b // bb, l // lb)

    if zprev is None:
        def body(x_ref, o_ref):
            o_ref[...] = jnp.transpose(x_ref[:, :, 0:e], (2, 0, 1))

        return pl.pallas_call(
            body, grid=grid, in_specs=[in_spec], out_specs=out_spec,
            out_shape=out_shape,
        )(emb)

    def body2(x_ref, z_ref, o_ref):
        o_ref[...] = jnp.transpose(x_ref[:, :, 0:e], (2, 0, 1))

    return pl.pallas_call(
        body2, grid=grid,
        in_specs=[in_spec, pl.BlockSpec(memory_space=pl.ANY)],
        out_specs=out_spec, out_shape=out_shape,
        input_output_aliases={1: 0},
    )(emb, zprev)


L_SPLIT = 96


def kernel(x, table):
    b, l = x.shape
    e = table.shape[1]
    idx_t = jnp.transpose(x).astype(jnp.int32)
    table_pad = jnp.pad(table, ((0, 0), (0, 128 - e)))
    emb1 = _gather_sc(table_pad, idx_t[:L_SPLIT].reshape(-1))
    emb2 = _gather_sc(table_pad, idx_t[L_SPLIT:].reshape(-1))
    z1 = _transpose_tc(emb1.reshape(L_SPLIT, b, 128), e, l, 0)
    z = _transpose_tc(emb2.reshape(l - L_SPLIT, b, 128), e, l,
                      L_SPLIT // 8, z1)
    return jnp.transpose(z, (2, 0, 1))
